# baseline (speedup 1.0000x reference)
import functools

import jax
import jax.numpy as jnp
from jax import lax
from jax.experimental import pallas as pl
from jax.experimental.pallas import tpu as pltpu
from jax.experimental.pallas import tpu_sc as plsc

NUM_EXPERTS = 64
TOP_K = 8
HIDDEN_SIZE = 2048
NUM_TOKENS = 16384

BT = 1024

NW = 32
RPW = NUM_TOKENS // NW


def _probs_body(h_ref, p_ref, o_ref):
    h = h_ref[...]
    p = p_ref[...]
    hn = jnp.maximum(jnp.sqrt(jnp.sum(h * h, axis=1, keepdims=True)), 1e-12)
    pn = jnp.maximum(jnp.sqrt(jnp.sum(p * p, axis=1, keepdims=True)), 1e-12)
    hb = (h / hn).astype(jnp.bfloat16)
    pb = (p / pn).astype(jnp.bfloat16)
    logits = jax.lax.dot_general(
        hb, pb, (((1,), (1,)), ((), ())), preferred_element_type=jnp.float32
    )
    m = jnp.max(logits, axis=1, keepdims=True)
    e = jnp.exp(logits - m)
    o_ref[...] = e / jnp.sum(e, axis=1, keepdims=True)


def _perm16(x, idx):
    dn = lax.GatherDimensionNumbers(
        offset_dims=(), collapsed_slice_dims=(0,), start_index_map=(0,)
    )
    return lax.gather(
        x, idx[:, None], dn, (1,), mode=lax.GatherScatterMode.PROMISE_IN_BOUNDS
    )


def _make_sc_topk():
    mesh = plsc.VectorSubcoreMesh(core_axis_name="c", subcore_axis_name="s")

    @functools.partial(
        pl.kernel,
        mesh=mesh,
        compiler_params=pltpu.CompilerParams(needs_layout_passes=False),
        out_type=[
            jax.ShapeDtypeStruct((NUM_TOKENS * TOP_K,), jnp.float32),
            jax.ShapeDtypeStruct((NUM_TOKENS * TOP_K,), jnp.int32),
        ],
        scratch_types=[
            pltpu.VMEM((RPW * NUM_EXPERTS,), jnp.float32),
            pltpu.VMEM((RPW * TOP_K,), jnp.float32),
            pltpu.VMEM((RPW * TOP_K,), jnp.int32),
        ],
    )
    def sc_topk(probs_hbm, w_hbm, i_hbm, probs_v, w_v, i_v):
        wid = lax.axis_index("s") * 2 + lax.axis_index("c")
        pltpu.sync_copy(
            probs_hbm.at[pl.ds(wid * (RPW * NUM_EXPERTS), RPW * NUM_EXPERTS)],
            probs_v,
        )

        lane = lax.iota(jnp.int32, 16)
        low8 = lane < 8
        rot8 = jnp.bitwise_and(lane + 8, 15)

        def topk_row(off):
            ks, vs = [], []
            for j in range(4):
                kj = probs_v[pl.ds(off + 16 * j, 16)]
                vj = lane + 16 * j
                kj, vj = plsc.sort_key_val(kj, vj, descending=True)
                ks.append(kj)
                vs.append(vj)

            def merge(ak, av, bk, bv):
                bk = lax.rev(bk, (0,))
                bv = lax.rev(bv, (0,))
                mk = jnp.where(low8, ak, bk)
                mv = jnp.where(low8, av, bv)
                return plsc.sort_key_val(mk, mv, descending=True)

            k01, v01 = merge(ks[0], vs[0], ks[1], vs[1])
            k23, v23 = merge(ks[2], vs[2], ks[3], vs[3])
            return merge(k01, v01, k23, v23)

        def pair(rp, _):
            ka, va = topk_row(rp * (2 * NUM_EXPERTS))
            kb, vb = topk_row(rp * (2 * NUM_EXPERTS) + NUM_EXPERTS)
            w_v[pl.ds(rp * 16, 16)] = jnp.where(low8, ka, _perm16(kb, rot8))
            i_v[pl.ds(rp * 16, 16)] = jnp.where(low8, va, _perm16(vb, rot8))
            return 0

        lax.fori_loop(0, RPW // 2, pair, 0)

        pltpu.sync_copy(w_v, w_hbm.at[pl.ds(wid * (RPW * TOP_K), RPW * TOP_K)])
        pltpu.sync_copy(i_v, i_hbm.at[pl.ds(wid * (RPW * TOP_K), RPW * TOP_K)])

    return sc_topk


_sc_topk = _make_sc_topk()


@jax.jit
def kernel(hidden_states, proto):
    grid = (NUM_TOKENS // BT,)
    probs = pl.pallas_call(
        _probs_body,
        grid=grid,
        in_specs=[
            pl.BlockSpec((BT, HIDDEN_SIZE), lambda t: (t, 0)),
            pl.BlockSpec((NUM_EXPERTS, HIDDEN_SIZE), lambda t: (0, 0)),
        ],
        out_specs=pl.BlockSpec((BT, NUM_EXPERTS), lambda t: (t, 0)),
        out_shape=jax.ShapeDtypeStruct((NUM_TOKENS, NUM_EXPERTS), jnp.float32),
    )(hidden_states, proto)
    w_flat, i_flat = _sc_topk(probs.reshape(-1))
    return (
        w_flat.reshape(NUM_TOKENS, TOP_K),
        i_flat.reshape(NUM_TOKENS, TOP_K),
    )


# speedup vs baseline: 1.1733x; 1.1733x over previous
import functools

import jax
import jax.numpy as jnp
from jax.experimental import pallas as pl
from jax.experimental.pallas import tpu as pltpu

NUM_EXPERTS = 64
TOP_K = 8
HIDDEN_SIZE = 2048
NUM_TOKENS = 16384

BT = 512


def _router_body(h_ref, p_ref, w_ref, i_ref):
    h = h_ref[...]
    p = p_ref[...]
    hn = jnp.maximum(jnp.sqrt(jnp.sum(h * h, axis=1, keepdims=True)), 1e-12)
    pn = jnp.maximum(jnp.sqrt(jnp.sum(p * p, axis=1, keepdims=True)), 1e-12)
    hb = (h / hn).astype(jnp.bfloat16)
    pb = (p / pn).astype(jnp.bfloat16)
    logits = jax.lax.dot_general(
        hb, pb, (((1,), (1,)), ((), ())), preferred_element_type=jnp.float32
    )
    m = jnp.max(logits, axis=1, keepdims=True)
    e = jnp.exp(logits - m)
    probs = e / jnp.sum(e, axis=1, keepdims=True)

    iota = jax.lax.broadcasted_iota(jnp.int32, probs.shape, 1).astype(jnp.float32)
    col8 = jax.lax.broadcasted_iota(jnp.int32, (probs.shape[0], TOP_K), 1).astype(
        jnp.float32
    )
    x = probs
    acc_w = jnp.zeros((probs.shape[0], TOP_K), jnp.float32)
    acc_i = jnp.zeros((probs.shape[0], TOP_K), jnp.float32)
    for k in range(TOP_K):
        mk = jnp.max(x, axis=1, keepdims=True)
        imf = jnp.min(
            jnp.where(x == mk, iota, float(NUM_EXPERTS)), axis=1, keepdims=True
        )
        acc_w = acc_w + jnp.where(col8 == float(k), mk, 0.0)
        acc_i = acc_i + jnp.where(col8 == float(k), imf, 0.0)
        x = jnp.where(iota == imf, -1.0, x)
    w_ref[...] = acc_w
    i_ref[...] = acc_i.astype(jnp.int32)


@jax.jit
def kernel(hidden_states, proto):
    grid = (NUM_TOKENS // BT,)
    return pl.pallas_call(
        _router_body,
        grid=grid,
        in_specs=[
            pl.BlockSpec((BT, HIDDEN_SIZE), lambda t: (t, 0)),
            pl.BlockSpec((NUM_EXPERTS, HIDDEN_SIZE), lambda t: (0, 0)),
        ],
        out_specs=[
            pl.BlockSpec((BT, TOP_K), lambda t: (t, 0)),
            pl.BlockSpec((BT, TOP_K), lambda t: (t, 0)),
        ],
        out_shape=[
            jax.ShapeDtypeStruct((NUM_TOKENS, TOP_K), jnp.float32),
            jax.ShapeDtypeStruct((NUM_TOKENS, TOP_K), jnp.int32),
        ],
    )(hidden_states, proto)


# speedup vs baseline: 1.7096x; 1.4571x over previous
import functools

import jax
import jax.numpy as jnp
from jax.experimental import pallas as pl

NUM_EXPERTS = 64
TOP_K = 8
HIDDEN_SIZE = 2048
NUM_TOKENS = 16384

BT = 1024


def _process(h, p):
    hn = jnp.maximum(jnp.sqrt(jnp.sum(h * h, axis=1, keepdims=True)), 1e-12)
    pn = jnp.maximum(jnp.sqrt(jnp.sum(p * p, axis=1, keepdims=True)), 1e-12)
    hb = (h / hn).astype(jnp.bfloat16)
    pb = (p / pn).astype(jnp.bfloat16)
    logits = jax.lax.dot_general(
        hb, pb, (((1,), (1,)), ((), ())), preferred_element_type=jnp.float32
    )
    lt = logits.T
    m = jnp.max(lt, axis=0, keepdims=True)
    e = jnp.exp(lt - m)
    probs = e / jnp.sum(e, axis=0, keepdims=True)

    iota = jax.lax.broadcasted_iota(jnp.int32, probs.shape, 0).astype(jnp.float32)
    row8 = jax.lax.broadcasted_iota(jnp.int32, (TOP_K, probs.shape[1]), 0).astype(
        jnp.float32
    )
    x = probs
    acc_w = jnp.zeros((TOP_K, probs.shape[1]), jnp.float32)
    acc_i = jnp.zeros((TOP_K, probs.shape[1]), jnp.float32)
    for k in range(TOP_K):
        mk = jnp.max(x, axis=0, keepdims=True)
        imf = jnp.min(
            jnp.where(x == mk, iota, float(NUM_EXPERTS)), axis=0, keepdims=True
        )
        acc_w = acc_w + jnp.where(row8 == float(k), mk, 0.0)
        acc_i = acc_i + jnp.where(row8 == float(k), imf, 0.0)
        x = jnp.where(iota == imf, -1.0, x)
    return acc_w.T, acc_i.T.astype(jnp.int32)


def _router_body(h1_ref, h2_ref, p_ref, w_ref, i_ref):
    p = p_ref[...]
    w1, i1 = _process(h1_ref[...], p)
    w_ref[0:BT, :] = w1
    i_ref[0:BT, :] = i1
    w2, i2 = _process(h2_ref[...], p)
    w_ref[BT : 2 * BT, :] = w2
    i_ref[BT : 2 * BT, :] = i2


@jax.jit
def kernel(hidden_states, proto):
    grid = (NUM_TOKENS // (2 * BT),)
    return pl.pallas_call(
        _router_body,
        grid=grid,
        in_specs=[
            pl.BlockSpec((BT, HIDDEN_SIZE), lambda t: (2 * t, 0)),
            pl.BlockSpec((BT, HIDDEN_SIZE), lambda t: (2 * t + 1, 0)),
            pl.BlockSpec((NUM_EXPERTS, HIDDEN_SIZE), lambda t: (0, 0)),
        ],
        out_specs=[
            pl.BlockSpec((2 * BT, TOP_K), lambda t: (t, 0)),
            pl.BlockSpec((2 * BT, TOP_K), lambda t: (t, 0)),
        ],
        out_shape=[
            jax.ShapeDtypeStruct((NUM_TOKENS, TOP_K), jnp.float32),
            jax.ShapeDtypeStruct((NUM_TOKENS, TOP_K), jnp.int32),
        ],
    )(hidden_states, hidden_states, proto)
